# Initial kernel scaffold; baseline (speedup 1.0000x reference)
import functools

import jax
import jax.numpy as jnp
from jax import lax
from jax.experimental import pallas as pl
from jax.experimental.pallas import tpu as pltpu
from jax.experimental.pallas import tpu_sc as plsc

_NC = 2
_NS = 16
_NW = _NC * _NS

_C = 125

_HL = 16


def _make_deg_kernel(n_nodes, n_edges):
    ept = n_edges // _NW
    nchunk = ept // _C
    rows_pt = n_nodes // _NS
    zchunks = rows_pt // _C

    mesh = plsc.VectorSubcoreMesh(
        core_axis_name="c", subcore_axis_name="s",
        num_cores=_NC, num_subcores=_NS)

    @functools.partial(
        pl.kernel,
        mesh=mesh,
        out_type=jax.ShapeDtypeStruct((_NC * n_nodes, _HL), jnp.float32),
        scratch_types=[
            pltpu.VMEM((nchunk, _C), jnp.int32),
            pltpu.VMEM((_C, _HL), jnp.float32),
            pltpu.VMEM_SHARED((n_nodes, _HL), jnp.float32),
        ],
    )
    def deg_kernel(dst3, ones_h, z16, degp, dst_v, ones_v, deg_sh):
        c = lax.axis_index("c")
        s = lax.axis_index("s")
        wid = c * _NS + s
        base = s * rows_pt

        pltpu.sync_copy(dst3.at[wid], dst_v)
        pltpu.sync_copy(ones_h, ones_v)
        for i in range(zchunks):
            pltpu.sync_copy(z16, deg_sh.at[pl.ds(base + i * _C, _C)])
        plsc.subcore_barrier()

        def chunk(j, carry):
            pltpu.sync_copy(ones_v, deg_sh.at[dst_v.at[j]], add=True)
            return carry

        lax.fori_loop(0, nchunk, chunk, 0)
        plsc.subcore_barrier()

        pltpu.sync_copy(
            deg_sh.at[pl.ds(base, rows_pt)],
            degp.at[pl.ds(c * n_nodes + base, rows_pt)])

    return deg_kernel


def _make_msg_kernel(n_nodes, n_edges, d_out):
    ept = n_edges // _NW
    nchunk = ept // _C
    rows_pt = n_nodes // _NS
    zchunks = rows_pt // _C

    mesh = plsc.VectorSubcoreMesh(
        core_axis_name="c", subcore_axis_name="s",
        num_cores=_NC, num_subcores=_NS)

    @functools.partial(
        pl.kernel,
        mesh=mesh,
        out_type=jax.ShapeDtypeStruct((_NC * n_nodes, d_out), jnp.float32),
        scratch_types=[
            pltpu.VMEM((nchunk, _C), jnp.int32),
            pltpu.VMEM((nchunk, _C), jnp.int32),
            pltpu.VMEM((_C, d_out), jnp.float32),
            pltpu.VMEM_SHARED((n_nodes, d_out), jnp.float32),
            pltpu.SemaphoreType.DMA,
        ],
    )
    def msg_kernel(y_h, src3, dst3, zrows, accp,
                   src_v, dst_v, rows_v, acc_sh, sem):
        c = lax.axis_index("c")
        s = lax.axis_index("s")
        wid = c * _NS + s
        base = s * rows_pt

        pltpu.sync_copy(src3.at[wid], src_v)
        pltpu.sync_copy(dst3.at[wid], dst_v)
        for i in range(zchunks):
            pltpu.sync_copy(zrows, acc_sh.at[pl.ds(base + i * _C, _C)])
        plsc.subcore_barrier()

        def chunk(j, carry):
            pltpu.async_copy(y_h.at[src_v.at[j]], rows_v, sem).wait()
            pltpu.sync_copy(rows_v, acc_sh.at[dst_v.at[j]], add=True)
            return carry

        lax.fori_loop(0, nchunk, chunk, 0)
        plsc.subcore_barrier()

        pltpu.sync_copy(
            acc_sh.at[pl.ds(base, rows_pt)],
            accp.at[pl.ds(c * n_nodes + base, rows_pt)])

    return msg_kernel


def _mm_body(x_ref, w_ref, dp_ref, y_ref):
    xw = jnp.dot(x_ref[...], w_ref[...], preferred_element_type=jnp.float32)
    d = dp_ref[...]
    deg = d[0, :, 0] + d[1, :, 0] + 1.0
    dinv = lax.rsqrt(deg)
    y_ref[...] = xw * dinv[:, None]


def _ep_body(a_ref, y_ref, dp_ref, b_ref, o_ref):
    a = a_ref[...]
    y = y_ref[...]
    d = dp_ref[...]
    deg = d[0, :, 0] + d[1, :, 0] + 1.0
    dinv = lax.rsqrt(deg)
    o_ref[...] = dinv[:, None] * (a[0] + a[1] + y) + b_ref[...]


def kernel(x, edge_index, W, b):
    n, d_in = x.shape
    d_out = W.shape[1]
    n_edges = edge_index.shape[1]

    src3 = edge_index[0].reshape(_NW, -1, _C)
    dst3 = edge_index[1].reshape(_NW, -1, _C)

    ones_h = jnp.ones((_C, _HL), jnp.float32)
    z16 = jnp.zeros((_C, _HL), jnp.float32)
    zrows = jnp.zeros((_C, d_out), jnp.float32)

    degp = _make_deg_kernel(n, n_edges)(dst3, ones_h, z16)
    degp = degp.reshape(_NC, n, _HL)

    blk = 1000
    grid = n // blk
    y = pl.pallas_call(
        _mm_body,
        grid=(grid,),
        in_specs=[
            pl.BlockSpec((blk, d_in), lambda i: (i, 0)),
            pl.BlockSpec((d_in, d_out), lambda i: (0, 0)),
            pl.BlockSpec((_NC, blk, _HL), lambda i: (0, i, 0)),
        ],
        out_specs=pl.BlockSpec((blk, d_out), lambda i: (i, 0)),
        out_shape=jax.ShapeDtypeStruct((n, d_out), jnp.float32),
    )(x, W, degp)

    accp = _make_msg_kernel(n, n_edges, d_out)(y, src3, dst3, zrows)
    accp = accp.reshape(_NC, n, d_out)

    out = pl.pallas_call(
        _ep_body,
        grid=(grid,),
        in_specs=[
            pl.BlockSpec((_NC, blk, d_out), lambda i: (0, i, 0)),
            pl.BlockSpec((blk, d_out), lambda i: (i, 0)),
            pl.BlockSpec((_NC, blk, _HL), lambda i: (0, i, 0)),
            pl.BlockSpec((1, d_out), lambda i: (0, 0)),
        ],
        out_specs=pl.BlockSpec((blk, d_out), lambda i: (i, 0)),
        out_shape=jax.ShapeDtypeStruct((n, d_out), jnp.float32),
    )(accp, y, degp, b.reshape(1, d_out))
    return out


# speedup vs baseline: 28.0450x; 28.0450x over previous
import functools

import jax
import jax.numpy as jnp
from jax import lax
from jax.experimental import pallas as pl
from jax.experimental.pallas import tpu as pltpu
from jax.experimental.pallas import tpu_sc as plsc

_NC = 2
_NS = 16
_NW = _NC * _NS

_C = 125

_ZC = 128

_HL = 16


def _make_deg_kernel(n_pad, n_edges):
    ept = n_edges // _NW
    nchunk = ept // _C
    rows_pt = n_pad // _NS
    zchunks = rows_pt // _ZC

    mesh = plsc.VectorSubcoreMesh(
        core_axis_name="c", subcore_axis_name="s",
        num_cores=_NC, num_subcores=_NS)

    @functools.partial(
        pl.kernel,
        mesh=mesh,
        out_type=jax.ShapeDtypeStruct((_NC * n_pad, _HL), jnp.float32),
        scratch_types=[
            pltpu.VMEM((nchunk, _C), jnp.int32),
            pltpu.VMEM((_C, _HL), jnp.float32),
            pltpu.VMEM_SHARED((n_pad, _HL), jnp.float32),
        ],
    )
    def deg_kernel(dst3, ones_h, z16, degp, dst_v, ones_v, deg_sh):
        c = lax.axis_index("c")
        s = lax.axis_index("s")
        wid = c * _NS + s
        base = s * rows_pt

        pltpu.sync_copy(dst3.at[wid], dst_v)
        pltpu.sync_copy(ones_h, ones_v)
        for i in range(zchunks):
            pltpu.sync_copy(z16, deg_sh.at[pl.ds(base + i * _ZC, _ZC)])
        plsc.subcore_barrier()

        def chunk(j, carry):
            pltpu.sync_copy(ones_v, deg_sh.at[dst_v.at[j]], add=True)
            return carry

        lax.fori_loop(0, nchunk, chunk, 0)
        plsc.subcore_barrier()

        pltpu.sync_copy(
            deg_sh.at[pl.ds(base, rows_pt)],
            degp.at[pl.ds(c * n_pad + base, rows_pt)])

    return deg_kernel


def _make_msg_kernel(n_pad, n_edges, d_out):
    ept = n_edges // _NW
    nchunk = ept // _C
    rows_pt = n_pad // _NS
    zchunks = rows_pt // _ZC

    mesh = plsc.VectorSubcoreMesh(
        core_axis_name="c", subcore_axis_name="s",
        num_cores=_NC, num_subcores=_NS)

    @functools.partial(
        pl.kernel,
        mesh=mesh,
        out_type=jax.ShapeDtypeStruct((_NC * n_pad, d_out), jnp.float32),
        scratch_types=[
            pltpu.VMEM((nchunk, _C), jnp.int32),
            pltpu.VMEM((nchunk, _C), jnp.int32),
            pltpu.VMEM((_C, d_out), jnp.float32),
            pltpu.VMEM_SHARED((n_pad, d_out), jnp.float32),
            pltpu.SemaphoreType.DMA,
        ],
    )
    def msg_kernel(y_h, src3, dst3, zrows, accp,
                   src_v, dst_v, rows_v, acc_sh, sem):
        c = lax.axis_index("c")
        s = lax.axis_index("s")
        wid = c * _NS + s
        base = s * rows_pt

        pltpu.sync_copy(src3.at[wid], src_v)
        pltpu.sync_copy(dst3.at[wid], dst_v)
        for i in range(zchunks):
            pltpu.sync_copy(zrows, acc_sh.at[pl.ds(base + i * _ZC, _ZC)])
        plsc.subcore_barrier()

        def chunk(j, carry):
            pltpu.async_copy(y_h.at[src_v.at[j]], rows_v, sem).wait()
            pltpu.sync_copy(rows_v, acc_sh.at[dst_v.at[j]], add=True)
            return carry

        lax.fori_loop(0, nchunk, chunk, 0)
        plsc.subcore_barrier()

        pltpu.sync_copy(
            acc_sh.at[pl.ds(base, rows_pt)],
            accp.at[pl.ds(c * n_pad + base, rows_pt)])

    return msg_kernel


def _mm_body(x_ref, w_ref, dp_ref, y_ref):
    xw = jnp.dot(x_ref[...], w_ref[...], preferred_element_type=jnp.float32)
    d = dp_ref[...]
    deg = d[0, :, 0] + d[1, :, 0] + 1.0
    dinv = lax.rsqrt(deg)
    y_ref[...] = xw * dinv[:, None]


def _ep_body(a_ref, y_ref, dp_ref, b_ref, o_ref):
    a = a_ref[...]
    y = y_ref[...]
    d = dp_ref[...]
    deg = d[0, :, 0] + d[1, :, 0] + 1.0
    dinv = lax.rsqrt(deg)
    o_ref[...] = dinv[:, None] * (a[0] + a[1] + y) + b_ref[...]


def kernel(x, edge_index, W, b):
    n, d_in = x.shape
    d_out = W.shape[1]
    n_edges = edge_index.shape[1]
    n_pad = (n + 127) // 128 * 128

    x_p = jnp.pad(x, ((0, n_pad - n), (0, 0)))
    src3 = edge_index[0].reshape(_NW, -1, _C)
    dst3 = edge_index[1].reshape(_NW, -1, _C)

    ones_h = jnp.ones((_C, _HL), jnp.float32)
    z16 = jnp.zeros((_ZC, _HL), jnp.float32)
    zrows = jnp.zeros((_ZC, d_out), jnp.float32)

    degp = _make_deg_kernel(n_pad, n_edges)(dst3, ones_h, z16)
    degp = degp.reshape(_NC, n_pad, _HL)

    blk = 1024
    grid = n_pad // blk
    y = pl.pallas_call(
        _mm_body,
        grid=(grid,),
        in_specs=[
            pl.BlockSpec((blk, d_in), lambda i: (i, 0)),
            pl.BlockSpec((d_in, d_out), lambda i: (0, 0)),
            pl.BlockSpec((_NC, blk, _HL), lambda i: (0, i, 0)),
        ],
        out_specs=pl.BlockSpec((blk, d_out), lambda i: (i, 0)),
        out_shape=jax.ShapeDtypeStruct((n_pad, d_out), jnp.float32),
    )(x_p, W, degp)

    accp = _make_msg_kernel(n_pad, n_edges, d_out)(y, src3, dst3, zrows)
    accp = accp.reshape(_NC, n_pad, d_out)

    out = pl.pallas_call(
        _ep_body,
        grid=(grid,),
        in_specs=[
            pl.BlockSpec((_NC, blk, d_out), lambda i: (0, i, 0)),
            pl.BlockSpec((blk, d_out), lambda i: (i, 0)),
            pl.BlockSpec((_NC, blk, _HL), lambda i: (0, i, 0)),
            pl.BlockSpec((1, d_out), lambda i: (0, 0)),
        ],
        out_specs=pl.BlockSpec((blk, d_out), lambda i: (i, 0)),
        out_shape=jax.ShapeDtypeStruct((n_pad, d_out), jnp.float32),
    )(accp, y, degp, b.reshape(1, d_out))
    return out[:n]


# speedup vs baseline: 30.2895x; 1.0800x over previous
import functools

import jax
import jax.numpy as jnp
from jax import lax
from jax.experimental import pallas as pl
from jax.experimental.pallas import tpu as pltpu
from jax.experimental.pallas import tpu_sc as plsc

_NC = 2
_NS = 16
_NW = _NC * _NS

_C = 125

_ZC = 128


def _make_deg_kernel(n_pad, n_edges):
    ept = n_edges // _NW
    nchunk = ept // _C
    rows_pt = n_pad // _NS

    mesh = plsc.VectorSubcoreMesh(
        core_axis_name="c", subcore_axis_name="s",
        num_cores=_NC, num_subcores=_NS)

    @functools.partial(
        pl.kernel,
        mesh=mesh,
        out_type=jax.ShapeDtypeStruct((_NC * n_pad,), jnp.float32),
        scratch_types=[
            pltpu.VMEM((nchunk, _C), jnp.int32),
            pltpu.VMEM((_C,), jnp.float32),
            pltpu.VMEM((rows_pt,), jnp.float32),
            pltpu.VMEM_SHARED((n_pad,), jnp.float32),
        ],
    )
    def deg_kernel(dst3, ones_h, degp, dst_v, ones_v, z_v, deg_sh):
        c = lax.axis_index("c")
        s = lax.axis_index("s")
        wid = c * _NS + s
        base = s * rows_pt

        pltpu.sync_copy(dst3.at[wid], dst_v)
        pltpu.sync_copy(ones_h, ones_v)
        for i in range(rows_pt // 16):
            z_v[pl.ds(i * 16, 16)] = jnp.zeros((16,), jnp.float32)
        pltpu.sync_copy(z_v, deg_sh.at[pl.ds(base, rows_pt)])
        plsc.subcore_barrier()

        def chunk(j, carry):
            pltpu.sync_copy(ones_v, deg_sh.at[dst_v.at[j]], add=True)
            return carry

        lax.fori_loop(0, nchunk, chunk, 0)
        plsc.subcore_barrier()

        pltpu.sync_copy(
            deg_sh.at[pl.ds(base, rows_pt)],
            degp.at[pl.ds(c * n_pad + base, rows_pt)])

    return deg_kernel


def _make_msg_kernel(n_pad, n_edges, d_out):
    ept = n_edges // _NW
    nchunk = ept // _C
    rows_pt = n_pad // _NS
    zchunks = rows_pt // _ZC

    mesh = plsc.VectorSubcoreMesh(
        core_axis_name="c", subcore_axis_name="s",
        num_cores=_NC, num_subcores=_NS)

    @functools.partial(
        pl.kernel,
        mesh=mesh,
        out_type=jax.ShapeDtypeStruct((_NC * n_pad, d_out), jnp.float32),
        scratch_types=[
            pltpu.VMEM((nchunk, _C), jnp.int32),
            pltpu.VMEM((nchunk, _C), jnp.int32),
            pltpu.VMEM((_C, d_out), jnp.float32),
            pltpu.VMEM_SHARED((n_pad, d_out), jnp.float32),
            pltpu.SemaphoreType.DMA,
        ],
    )
    def msg_kernel(y_h, src3, dst3, zrows, accp,
                   src_v, dst_v, rows_v, acc_sh, sem):
        c = lax.axis_index("c")
        s = lax.axis_index("s")
        wid = c * _NS + s
        base = s * rows_pt

        pltpu.sync_copy(src3.at[wid], src_v)
        pltpu.sync_copy(dst3.at[wid], dst_v)
        for i in range(zchunks):
            pltpu.sync_copy(zrows, acc_sh.at[pl.ds(base + i * _ZC, _ZC)])
        plsc.subcore_barrier()

        def chunk(j, carry):
            pltpu.async_copy(y_h.at[src_v.at[j]], rows_v, sem).wait()
            pltpu.sync_copy(rows_v, acc_sh.at[dst_v.at[j]], add=True)
            return carry

        lax.fori_loop(0, nchunk, chunk, 0)
        plsc.subcore_barrier()

        pltpu.sync_copy(
            acc_sh.at[pl.ds(base, rows_pt)],
            accp.at[pl.ds(c * n_pad + base, rows_pt)])

    return msg_kernel


def _mm_body(x_ref, w_ref, dp_ref, y_ref):
    xw = jnp.dot(x_ref[...], w_ref[...], preferred_element_type=jnp.float32)
    d = dp_ref[...]
    deg = d[0] + d[1] + 1.0
    dinv = lax.rsqrt(deg)
    y_ref[...] = xw * dinv[:, None]


def _ep_body(a_ref, y_ref, dp_ref, b_ref, o_ref):
    a = a_ref[...]
    y = y_ref[...]
    d = dp_ref[...]
    deg = d[0] + d[1] + 1.0
    dinv = lax.rsqrt(deg)
    o_ref[...] = dinv[:, None] * (a[0] + a[1] + y) + b_ref[...]


def kernel(x, edge_index, W, b):
    n, d_in = x.shape
    d_out = W.shape[1]
    n_edges = edge_index.shape[1]
    n_pad = (n + 2047) // 2048 * 2048

    x_p = jnp.pad(x, ((0, n_pad - n), (0, 0)))
    src3 = edge_index[0].reshape(_NW, -1, _C)
    dst3 = edge_index[1].reshape(_NW, -1, _C)

    ones_h = jnp.ones((_C,), jnp.float32)
    zrows = jnp.zeros((_ZC, d_out), jnp.float32)

    degp = _make_deg_kernel(n_pad, n_edges)(dst3, ones_h)
    degp = degp.reshape(_NC, n_pad)

    blk = 1024
    grid = n_pad // blk
    y = pl.pallas_call(
        _mm_body,
        grid=(grid,),
        in_specs=[
            pl.BlockSpec((blk, d_in), lambda i: (i, 0)),
            pl.BlockSpec((d_in, d_out), lambda i: (0, 0)),
            pl.BlockSpec((_NC, blk), lambda i: (0, i)),
        ],
        out_specs=pl.BlockSpec((blk, d_out), lambda i: (i, 0)),
        out_shape=jax.ShapeDtypeStruct((n_pad, d_out), jnp.float32),
    )(x_p, W, degp)

    accp = _make_msg_kernel(n_pad, n_edges, d_out)(y, src3, dst3, zrows)
    accp = accp.reshape(_NC, n_pad, d_out)

    out = pl.pallas_call(
        _ep_body,
        grid=(grid,),
        in_specs=[
            pl.BlockSpec((_NC, blk, d_out), lambda i: (0, i, 0)),
            pl.BlockSpec((blk, d_out), lambda i: (i, 0)),
            pl.BlockSpec((_NC, blk), lambda i: (0, i)),
            pl.BlockSpec((1, d_out), lambda i: (0, 0)),
        ],
        out_specs=pl.BlockSpec((blk, d_out), lambda i: (i, 0)),
        out_shape=jax.ShapeDtypeStruct((n_pad, d_out), jnp.float32),
    )(accp, y, degp, b.reshape(1, d_out))
    return out[:n]
